# Initial kernel scaffold; baseline (speedup 1.0000x reference)
import jax
import jax.numpy as jnp
from jax.experimental import pallas as pl


def kernel(outA, outB, matchA, matchB, nonMatchA, nonMatchB):
    raise NotImplementedError("write your pallas kernel here")


# speedup vs baseline: 1.3077x; 1.3077x over previous
import functools

import jax
import jax.numpy as jnp
from jax import lax
from jax.experimental import pallas as pl
from jax.experimental.pallas import tpu as pltpu
from jax.experimental.pallas import tpu_sc as plsc

_MARGIN = 0.5
_NON_MATCH_LOSS_WEIGHT = 1.0
_LANES = 16


def _sc_geometry():
    try:
        info = plsc.get_sparse_core_info()
        return info.num_cores, info.num_subcores
    except Exception:
        return 2, 16


@functools.partial(jax.jit, static_argnums=(6, 7, 8, 9))
def _partials(a2, b2, mA, mB, nA, nB, B, N, D, M):
    NC, NS = _sc_geometry()
    NW = NC * NS
    PW = M // NW
    CH = min(PW, 128)
    NCH = PW // CH
    mesh = plsc.VectorSubcoreMesh(core_axis_name="c", subcore_axis_name="s",
                                  num_cores=NC, num_subcores=NS)

    def body(a_hbm, b_hbm, mA_hbm, mB_hbm, nA_hbm, nB_hbm, out_hbm,
             idxa_v, idxb_v, rowsa_v, rowsb_v, res_v, sema, semb):
        wid = lax.axis_index("s") * NC + lax.axis_index("c")
        base = wid * PW
        for b in range(B):
            off = jnp.int32(b * N)
            for si, (iA_hbm, iB_hbm) in enumerate(
                    ((mA_hbm, mB_hbm), (nA_hbm, nB_hbm))):
                acc0 = jnp.zeros((_LANES,), jnp.float32)
                acc1 = jnp.zeros((_LANES,), jnp.float32)
                for ch in range(NCH):
                    cbase = base + ch * CH
                    pltpu.sync_copy(iA_hbm.at[b, pl.ds(cbase, CH)], idxa_v)
                    pltpu.sync_copy(iB_hbm.at[b, pl.ds(cbase, CH)], idxb_v)
                    for i in range(CH // _LANES):
                        sl = pl.ds(i * _LANES, _LANES)
                        idxa_v[sl] = idxa_v[sl] + off
                        idxb_v[sl] = idxb_v[sl] + off
                    cpa = pltpu.async_copy(a_hbm.at[idxa_v], rowsa_v, sema)
                    cpb = pltpu.async_copy(b_hbm.at[idxb_v], rowsb_v, semb)
                    cpa.wait()
                    cpb.wait()
                    if si == 0:
                        def mbody(r, acc):
                            for cc in range(D // _LANES):
                                sl = pl.ds(cc * _LANES, _LANES)
                                d = rowsa_v[r, sl] - rowsb_v[r, sl]
                                acc = acc + d * d
                            return acc
                        acc0 = lax.fori_loop(0, CH, mbody, acc0)
                    else:
                        def nbody(r, carry):
                            s0, s1 = carry
                            for cc in range(D // _LANES):
                                sl = pl.ds(cc * _LANES, _LANES)
                                d = rowsa_v[r, sl] - rowsb_v[r, sl]
                                t = _MARGIN - d * d
                                pos = t > 0.0
                                s0 = s0 + jnp.where(pos, t, 0.0)
                                s1 = s1 + jnp.where(pos, 1.0, 0.0)
                            return s0, s1
                        acc0, acc1 = lax.fori_loop(0, CH, nbody, (acc0, acc1))
                if si == 0:
                    res_v[b, 0] = acc0
                else:
                    res_v[b, 1] = acc0
                    res_v[b, 2] = acc1
        pltpu.sync_copy(res_v, out_hbm.at[wid])

    call = pl.kernel(
        body,
        out_type=jax.ShapeDtypeStruct((NW, B, 3, _LANES), jnp.float32),
        mesh=mesh,
        scratch_types=[
            pltpu.VMEM((CH,), jnp.int32),
            pltpu.VMEM((CH,), jnp.int32),
            pltpu.VMEM((CH, D), jnp.float32),
            pltpu.VMEM((CH, D), jnp.float32),
            pltpu.VMEM((B, 3, _LANES), jnp.float32),
            pltpu.SemaphoreType.DMA,
            pltpu.SemaphoreType.DMA,
        ],
        compiler_params=pltpu.CompilerParams(use_tc_tiling_on_sc=False),
    )
    return call(a2, b2, mA, mB, nA, nB)


def kernel(outA, outB, matchA, matchB, nonMatchA, nonMatchB):
    B, N, D = outA.shape
    M = matchA.shape[1]
    a2 = outA.reshape(B * N, D)
    b2 = outB.reshape(B * N, D)
    mA = matchA.astype(jnp.int32)
    mB = matchB.astype(jnp.int32)
    nA = nonMatchA.astype(jnp.int32)
    nB = nonMatchB.astype(jnp.int32)
    parts = _partials(a2, b2, mA, mB, nA, nB, B, N, D, M)
    sums = jnp.sum(parts, axis=(0, 3))
    match_loss = jnp.sum(sums[:, 0]) / M
    non_match_loss = _NON_MATCH_LOSS_WEIGHT * jnp.sum(sums[:, 1] / sums[:, 2])
    return (match_loss + non_match_loss, match_loss, non_match_loss)
